# baseline (speedup 1.0000x reference)
import jax
import jax.numpy as jnp
from jax.experimental import pallas as pl
from jax.experimental.pallas import tpu as pltpu

_TB = 128
_TH = 1024


def _round_up(x, m):
    return (x + m - 1) // m * m


def _conv_kernel(xin_ref, r1_ref, b1f_ref, r2_ref, b2f_ref, o_ref):
    TB = xin_ref.shape[0]
    f32 = jnp.float32
    xin = jnp.transpose(xin_ref[...], (1, 0, 2))
    hi0 = jnp.concatenate(
        [jnp.zeros((1, TB, 32), xin.dtype), xin[:, :, 96:128]], axis=0)
    xq = jnp.concatenate(
        [xin[:, :, 0:96], jnp.zeros((1, TB, 96), xin.dtype)], axis=0)
    xga = jnp.concatenate([hi0, xq], axis=2)

    lhs_o = xga.reshape(8 * TB, 128)
    hi = jnp.concatenate(
        [jnp.zeros((1, TB, 64), xga.dtype), xga[0:7, :, 64:128]], axis=0)
    xsh = jnp.concatenate([hi, xga[:, :, 0:64]], axis=2)
    lhs_e = xsh.reshape(8 * TB, 128)

    r1 = r1_ref[...]
    co = jnp.dot(lhs_o, r1, preferred_element_type=f32)
    ce = jnp.dot(lhs_e, r1, preferred_element_type=f32)

    b1f = b1f_ref[...]
    f8 = jnp.float8_e4m3fn
    zrow = jnp.zeros((1, TB, 256), f8)

    def pool1(c, pad_row):
        m = jnp.maximum(jnp.maximum(c[:, 0:256], c[:, 256:512]),
                        jnp.maximum(c[:, 512:768], c[:, 768:1024]))
        m = m.reshape(8, TB, 256)
        row = jax.lax.broadcasted_iota(jnp.int32, (8, TB, 1), 0)
        p = jnp.where(row == pad_row, 0.0, m + b1f).astype(f8)
        return jnp.concatenate([p, zrow], axis=0)

    p1o = pool1(co, 7)
    p1e = pool1(ce, 0)

    L = [(p1e if t2 % 2 == 0 else p1o)[t2 // 2:t2 // 2 + 8]
         .reshape(8 * TB, 256) for t2 in range(4)]
    r2 = r2_ref[...]
    cA = jnp.dot(L[0], r2[0], preferred_element_type=f32)
    cB = jnp.dot(L[1], r2[0], preferred_element_type=f32)
    for kh in (1, 2):
        cA = cA + jnp.dot(L[kh], r2[kh], preferred_element_type=f32)
        cB = cB + jnp.dot(L[kh + 1], r2[kh], preferred_element_type=f32)
    pooled2 = jnp.maximum(jnp.maximum(cA[:, 0:256], cA[:, 256:512]),
                          jnp.maximum(cB[:, 0:256], cB[:, 256:512]))
    pooled2 = (pooled2 + b2f_ref[...]).astype(jnp.bfloat16)
    o_ref[...] = pooled2.reshape(8, TB, 256)


def _head_kernel(x_ref, w1_ref, fb1_ref, w2_ref, fb2_ref, o_ref):
    TB = x_ref.shape[1]
    f32 = jnp.float32
    x = x_ref[...]
    d = [jnp.dot(x[qh], w1_ref[qh], preferred_element_type=f32)
         for qh in range(8)]
    h = ((d[0] + d[1]) + (d[2] + d[3])) + ((d[4] + d[5]) + (d[6] + d[7]))
    h = jnp.maximum(h + fb1_ref[...], 0.0).astype(jnp.bfloat16)
    y = jnp.dot(h, w2_ref[...], preferred_element_type=f32) + fb2_ref[...]
    z = y - jnp.max(y, axis=1, keepdims=True)
    ez = jnp.exp(z).astype(jnp.bfloat16)
    ones = jnp.ones((128, 128), jnp.bfloat16)
    s = jnp.dot(ez, ones, preferred_element_type=f32)[:, 0:1]
    o_ref[...] = z - jnp.log(s)


def _build_weights(m1, m2, fw1p):
    bf16 = jnp.bfloat16
    m1 = jnp.pad(m1[:, :, 1:, :], ((0, 0), (0, 0), (0, 1), (0, 0)))
    a0 = jnp.pad(m1, ((0, 1), (0, 0), (0, 0), (0, 0)))
    a1 = jnp.pad(m1, ((1, 0), (0, 0), (0, 0), (0, 0)))
    r1 = jnp.concatenate([a0.transpose(0, 2, 1, 3),
                          a1.transpose(0, 2, 1, 3)], axis=2)
    r1 = r1.reshape(128, 1024).astype(bf16)
    r2 = m2.transpose(0, 2, 1, 3).reshape(3, 256, 512).astype(jnp.float8_e4m3fn)
    w1 = jnp.pad(fw1p[:980].reshape(7, 7, 20, 128),
                 ((0, 1), (0, 1), (0, 12), (0, 0)))
    w1 = w1.reshape(8, 256, 128).astype(bf16)
    return r1, r2, w1


def kernel(x_nchw, m1, b1f, m2, b2f, fw1p, fb1r, fw2p, fb2p):
    B = x_nchw.shape[0]
    x = x_nchw.reshape(B, 28, 28)
    xin = jnp.pad(x, ((0, 0), (0, 0), (0, 4))).astype(jnp.bfloat16)
    xin = xin.reshape(B, 7, 128)

    Bp = _round_up(B, _TH)
    if Bp != B:
        xin = jnp.pad(xin, ((0, Bp - B), (0, 0), (0, 0)))

    r1, r2, w1 = _build_weights(m1, m2, fw1p)
    w2 = fw2p.astype(jnp.bfloat16)

    feat = pl.pallas_call(
        _conv_kernel,
        out_shape=jax.ShapeDtypeStruct((8, Bp, 256), jnp.bfloat16),
        grid=(Bp // _TB,),
        in_specs=[
            pl.BlockSpec((_TB, 7, 128), lambda r: (r, 0, 0)),
            pl.BlockSpec((128, 1024), lambda r: (0, 0)),
            pl.BlockSpec((1, 256), lambda r: (0, 0)),
            pl.BlockSpec((3, 256, 512), lambda r: (0, 0, 0)),
            pl.BlockSpec((1, 256), lambda r: (0, 0)),
        ],
        out_specs=pl.BlockSpec((8, _TB, 256), lambda r: (0, r, 0)),
        compiler_params=pltpu.CompilerParams(
            dimension_semantics=("parallel",)),
    )(xin, r1, b1f, r2, b2f)

    TH = _TH
    out = pl.pallas_call(
        _head_kernel,
        out_shape=jax.ShapeDtypeStruct((Bp, 128), jnp.float32),
        grid=(Bp // TH,),
        in_specs=[
            pl.BlockSpec((8, TH, 256), lambda r: (0, r, 0)),
            pl.BlockSpec((8, 256, 128), lambda r: (0, 0, 0)),
            pl.BlockSpec((1, 128), lambda r: (0, 0)),
            pl.BlockSpec((128, 128), lambda r: (0, 0)),
            pl.BlockSpec((1, 128), lambda r: (0, 0)),
        ],
        out_specs=pl.BlockSpec((TH, 128), lambda r: (r, 0)),
        compiler_params=pltpu.CompilerParams(
            dimension_semantics=("parallel",)),
    )(feat, w1, fb1r, w2, fb2p)
    return out[:B, :10]


# speedup vs baseline: 1.0809x; 1.0809x over previous
import jax
import jax.numpy as jnp
from jax.experimental import pallas as pl
from jax.experimental.pallas import tpu as pltpu

_TB = 128
_TH = 1024


def _round_up(x, m):
    return (x + m - 1) // m * m


def _conv_kernel(xin_ref, r1_ref, b1f_ref, r2_ref, b2f_ref, o_ref):
    TB = xin_ref.shape[0]
    f32 = jnp.float32
    xin = jnp.transpose(xin_ref[...], (1, 0, 2))
    hi0 = jnp.concatenate(
        [jnp.zeros((1, TB, 32), xin.dtype), xin[:, :, 96:128]], axis=0)
    xq = jnp.concatenate(
        [xin[:, :, 0:96], jnp.zeros((1, TB, 96), xin.dtype)], axis=0)
    xga = jnp.concatenate([hi0, xq], axis=2)

    lhs_o = xga.reshape(8 * TB, 128)
    hi = jnp.concatenate(
        [jnp.zeros((1, TB, 64), xga.dtype), xga[0:7, :, 64:128]], axis=0)
    xsh = jnp.concatenate([hi, xga[:, :, 0:64]], axis=2)
    lhs_e = xsh.reshape(8 * TB, 128)

    r1 = r1_ref[...]
    co = jnp.dot(lhs_o, r1, preferred_element_type=f32)
    ce = jnp.dot(lhs_e, r1, preferred_element_type=f32)

    b1f = b1f_ref[...]
    f8 = jnp.float8_e4m3fn
    zrow = jnp.zeros((1, TB, 256), f8)

    def pool1(c, pad_row):
        m = jnp.maximum(jnp.maximum(c[:, 0:256], c[:, 256:512]),
                        jnp.maximum(c[:, 512:768], c[:, 768:1024]))
        m = m.reshape(8, TB, 256)
        row = jax.lax.broadcasted_iota(jnp.int32, (8, TB, 1), 0)
        p = jnp.where(row == pad_row, 0.0, m + b1f).astype(f8)
        return jnp.concatenate([p, zrow], axis=0)

    p1o = pool1(co, 7)
    p1e = pool1(ce, 0)

    L = [(p1e if t2 % 2 == 0 else p1o)[t2 // 2:t2 // 2 + 8]
         .reshape(8 * TB, 256) for t2 in range(4)]
    r2 = r2_ref[...]
    cA = jnp.dot(L[0], r2[0], preferred_element_type=f32)
    cB = jnp.dot(L[1], r2[0], preferred_element_type=f32)
    for kh in (1, 2):
        cA = cA + jnp.dot(L[kh], r2[kh], preferred_element_type=f32)
        cB = cB + jnp.dot(L[kh + 1], r2[kh], preferred_element_type=f32)
    pooled2 = jnp.maximum(jnp.maximum(cA[:, 0:256], cA[:, 256:512]),
                          jnp.maximum(cB[:, 0:256], cB[:, 256:512]))
    pooled2 = (pooled2 + b2f_ref[...]).astype(jnp.bfloat16)
    o_ref[...] = pooled2.reshape(8, TB, 256)


def _head_kernel(x_ref, w1_ref, fb1_ref, w2_ref, fb2_ref, o_ref):
    TB = x_ref.shape[1]
    f32 = jnp.float32
    x = x_ref[...]
    d = [jnp.dot(x[qh], w1_ref[qh], preferred_element_type=f32)
         for qh in range(8)]
    h = ((d[0] + d[1]) + (d[2] + d[3])) + ((d[4] + d[5]) + (d[6] + d[7]))
    h = jnp.maximum(h + fb1_ref[...], 0.0).astype(jnp.bfloat16)
    y = jnp.dot(h, w2_ref[...], preferred_element_type=f32) + fb2_ref[...]
    z = y - jnp.max(y, axis=1, keepdims=True)
    ez = jnp.exp(z).astype(jnp.bfloat16)
    ones = jnp.ones((128, 128), jnp.bfloat16)
    s = jnp.dot(ez, ones, preferred_element_type=f32)[:, 0:1]
    o_ref[...] = z - jnp.log(s)


def _build_weights(m1, m2, fw1p):
    bf16 = jnp.bfloat16
    m1 = jnp.pad(m1[:, :, 1:, :], ((0, 0), (0, 0), (0, 1), (0, 0)))
    a0 = jnp.pad(m1, ((0, 1), (0, 0), (0, 0), (0, 0)))
    a1 = jnp.pad(m1, ((1, 0), (0, 0), (0, 0), (0, 0)))
    r1 = jnp.concatenate([a0.transpose(0, 2, 1, 3),
                          a1.transpose(0, 2, 1, 3)], axis=2)
    r1 = r1.reshape(128, 1024).astype(jnp.float8_e4m3fn)
    r2 = m2.transpose(0, 2, 1, 3).reshape(3, 256, 512).astype(jnp.float8_e4m3fn)
    w1 = jnp.pad(fw1p[:980].reshape(7, 7, 20, 128),
                 ((0, 1), (0, 1), (0, 12), (0, 0)))
    w1 = w1.reshape(8, 256, 128).astype(bf16)
    return r1, r2, w1


def kernel(x_nchw, m1, b1f, m2, b2f, fw1p, fb1r, fw2p, fb2p):
    B = x_nchw.shape[0]
    x = x_nchw.reshape(B, 28, 28)
    xin = jnp.pad(x, ((0, 0), (0, 0), (0, 4))).astype(jnp.float8_e4m3fn)
    xin = xin.reshape(B, 7, 128)

    Bp = _round_up(B, _TH)
    if Bp != B:
        xin = jnp.pad(xin, ((0, Bp - B), (0, 0), (0, 0)))

    r1, r2, w1 = _build_weights(m1, m2, fw1p)
    w2 = fw2p.astype(jnp.bfloat16)

    feat = pl.pallas_call(
        _conv_kernel,
        out_shape=jax.ShapeDtypeStruct((8, Bp, 256), jnp.bfloat16),
        grid=(Bp // _TB,),
        in_specs=[
            pl.BlockSpec((_TB, 7, 128), lambda r: (r, 0, 0)),
            pl.BlockSpec((128, 1024), lambda r: (0, 0)),
            pl.BlockSpec((1, 256), lambda r: (0, 0)),
            pl.BlockSpec((3, 256, 512), lambda r: (0, 0, 0)),
            pl.BlockSpec((1, 256), lambda r: (0, 0)),
        ],
        out_specs=pl.BlockSpec((8, _TB, 256), lambda r: (0, r, 0)),
        compiler_params=pltpu.CompilerParams(
            dimension_semantics=("parallel",)),
    )(xin, r1, b1f, r2, b2f)

    TH = _TH
    out = pl.pallas_call(
        _head_kernel,
        out_shape=jax.ShapeDtypeStruct((Bp, 128), jnp.float32),
        grid=(Bp // TH,),
        in_specs=[
            pl.BlockSpec((8, TH, 256), lambda r: (0, r, 0)),
            pl.BlockSpec((8, 256, 128), lambda r: (0, 0, 0)),
            pl.BlockSpec((1, 128), lambda r: (0, 0)),
            pl.BlockSpec((128, 128), lambda r: (0, 0)),
            pl.BlockSpec((1, 128), lambda r: (0, 0)),
        ],
        out_specs=pl.BlockSpec((TH, 128), lambda r: (r, 0)),
        compiler_params=pltpu.CompilerParams(
            dimension_semantics=("parallel",)),
    )(feat, w1, fb1r, w2, fb2p)
    return out[:B, :10]
